# Initial kernel scaffold; baseline (speedup 1.0000x reference)
import jax
import jax.numpy as jnp
from jax.experimental import pallas as pl


def kernel(input_ids, segment_ids, token_table, position_table, segment_table, ln_gamma, ln_beta):
    raise NotImplementedError("write your pallas kernel here")


# speedup vs baseline: 10.7235x; 10.7235x over previous
import functools

import jax
import jax.numpy as jnp
from jax import lax
from jax.experimental import pallas as pl
from jax.experimental.pallas import tpu as pltpu
from jax.experimental.pallas import tpu_sc as plsc

EPS = 1e-5


def _sc_gather(table, idx, chunk=640):
    n = idx.shape[0]
    d = table.shape[1]
    info = plsc.get_sparse_core_info()
    nc, ns = info.num_cores, info.num_subcores
    nw = nc * ns
    per_w = n // nw
    n_chunks = per_w // chunk
    assert per_w % chunk == 0 and n % nw == 0

    mesh = plsc.VectorSubcoreMesh(core_axis_name="c", subcore_axis_name="s")

    @functools.partial(
        pl.kernel,
        mesh=mesh,
        out_type=jax.ShapeDtypeStruct((n, d), jnp.float32),
        scratch_types=[
            pltpu.VMEM((chunk,), jnp.int32),
            pltpu.VMEM((chunk, d), jnp.float32),
            pltpu.SemaphoreType.DMA,
        ],
    )
    def k(table_hbm, idx_hbm, out_hbm, idx_v, rows_v, sem):
        wid = lax.axis_index("s") * nc + lax.axis_index("c")
        base = wid * per_w

        def body(c, carry):
            off = base + c * chunk
            pltpu.sync_copy(idx_hbm.at[pl.ds(off, chunk)], idx_v)
            pltpu.async_copy(table_hbm.at[idx_v], rows_v, sem).wait()
            pltpu.sync_copy(rows_v, out_hbm.at[pl.ds(off, chunk)])
            return carry

        lax.fori_loop(0, n_chunks, body, 0)

    return k(table, idx)


def _tc_epilogue(gathered, seg_ids, pos_tab, seg_tab, gamma, beta):
    b, l, d = gathered.shape
    blk = 16
    grid = (b // blk,)

    def body(g_ref, s_ref, p_ref, st_ref, ga_ref, be_ref, o_ref):
        x = g_ref[...]
        pos = p_ref[...]
        seg = s_ref[...]
        st = st_ref[...]
        emb = x + pos[None, :, :] + jnp.where(
            (seg[..., None] == 0), st[0][None, None, :], st[1][None, None, :]
        )
        mean = jnp.mean(emb, axis=-1, keepdims=True)
        cent = emb - mean
        var = jnp.mean(cent * cent, axis=-1, keepdims=True)
        normed = cent * lax.rsqrt(var + EPS)
        o_ref[...] = normed * ga_ref[0][None, None, :] + be_ref[0][None, None, :]

    return pl.pallas_call(
        body,
        grid=grid,
        in_specs=[
            pl.BlockSpec((blk, l, d), lambda i: (i, 0, 0)),
            pl.BlockSpec((blk, l), lambda i: (i, 0)),
            pl.BlockSpec((l, d), lambda i: (0, 0)),
            pl.BlockSpec((2, d), lambda i: (0, 0)),
            pl.BlockSpec((1, d), lambda i: (0, 0)),
            pl.BlockSpec((1, d), lambda i: (0, 0)),
        ],
        out_specs=pl.BlockSpec((blk, l, d), lambda i: (i, 0, 0)),
        out_shape=jax.ShapeDtypeStruct((b, l, d), jnp.float32),
    )(gathered, seg_ids, pos_tab, seg_tab, gamma, beta)


def kernel(input_ids, segment_ids, token_table, position_table, segment_table,
           ln_gamma, ln_beta):
    b, l = input_ids.shape
    d = token_table.shape[1]
    flat_ids = input_ids.reshape(b * l).astype(jnp.int32)
    gathered = _sc_gather(token_table, flat_ids)
    return _tc_epilogue(
        gathered.reshape(b, l, d),
        segment_ids.astype(jnp.int32),
        position_table[:l],
        segment_table,
        ln_gamma.reshape(1, d),
        ln_beta.reshape(1, d),
    )


# speedup vs baseline: 10.8008x; 1.0072x over previous
import functools

import jax
import jax.numpy as jnp
from jax import lax
from jax.experimental import pallas as pl
from jax.experimental.pallas import tpu as pltpu
from jax.experimental.pallas import tpu_sc as plsc

EPS = 1e-5


def _sc_gather(table, idx, chunk=256, nbuf=3):
    n = idx.shape[0]
    d = table.shape[1]
    info = plsc.get_sparse_core_info()
    nc, ns = info.num_cores, info.num_subcores
    nw = nc * ns
    per_w = n // nw
    n_chunks = per_w // chunk
    assert per_w % chunk == 0 and n % nw == 0

    mesh = plsc.VectorSubcoreMesh(core_axis_name="c", subcore_axis_name="s")

    @functools.partial(
        pl.kernel,
        mesh=mesh,
        out_type=jax.ShapeDtypeStruct((n, d), jnp.float32),
        scratch_types=[
            pltpu.VMEM((per_w,), jnp.int32),
            pltpu.VMEM((nbuf, chunk, d), jnp.float32),
            pltpu.SemaphoreType.DMA,
            [pltpu.SemaphoreType.DMA] * nbuf,
            [pltpu.SemaphoreType.DMA] * nbuf,
        ],
    )
    def k(table_hbm, idx_hbm, out_hbm, idx_v, rows_v, isem, gsems, wsems):
        wid = lax.axis_index("s") * nc + lax.axis_index("c")
        base = wid * per_w
        pltpu.async_copy(idx_hbm.at[pl.ds(base, per_w)], idx_v, isem).wait()

        def g_start(c, b):
            pltpu.async_copy(
                table_hbm.at[idx_v.at[pl.ds(c * chunk, chunk)]],
                rows_v.at[b], gsems[b])

        for b in range(min(nbuf, n_chunks)):
            g_start(b, b)
        for c in range(n_chunks):
            b = c % nbuf
            pltpu.make_async_copy(
                table_hbm.at[idx_v.at[pl.ds(c * chunk, chunk)]],
                rows_v.at[b], gsems[b]).wait()
            w = pltpu.async_copy(
                rows_v.at[b], out_hbm.at[pl.ds(base + c * chunk, chunk)],
                wsems[b])
            if c + nbuf < n_chunks:
                w.wait()
                g_start(c + nbuf, b)
        for c in range(max(0, n_chunks - nbuf), n_chunks):
            b = c % nbuf
            pltpu.make_async_copy(
                rows_v.at[b], out_hbm.at[pl.ds(base + c * chunk, chunk)],
                wsems[b]).wait()

    return k(table, idx)


def _tc_epilogue(gathered, seg_ids, pos_tab, seg_tab, gamma, beta):
    b, l, d = gathered.shape
    blk = 16
    grid = (b // blk,)

    def body(g_ref, s_ref, p_ref, st_ref, ga_ref, be_ref, o_ref):
        x = g_ref[...]
        segf = s_ref[...]
        st = st_ref[...]
        p0 = p_ref[...] + st[0][None, :]
        sd = st[1] - st[0]
        emb = x + p0[None, :, :] + segf[..., None] * sd[None, None, :]
        s1 = jnp.sum(emb, axis=-1, keepdims=True)
        s2 = jnp.sum(emb * emb, axis=-1, keepdims=True)
        mean = s1 * (1.0 / d)
        var = s2 * (1.0 / d) - mean * mean
        r = lax.rsqrt(var + EPS)
        o_ref[...] = (emb - mean) * r * ga_ref[0][None, None, :] + be_ref[0][None, None, :]

    return pl.pallas_call(
        body,
        grid=grid,
        in_specs=[
            pl.BlockSpec((blk, l, d), lambda i: (i, 0, 0)),
            pl.BlockSpec((blk, l), lambda i: (i, 0)),
            pl.BlockSpec((l, d), lambda i: (0, 0)),
            pl.BlockSpec((2, d), lambda i: (0, 0)),
            pl.BlockSpec((1, d), lambda i: (0, 0)),
            pl.BlockSpec((1, d), lambda i: (0, 0)),
        ],
        out_specs=pl.BlockSpec((blk, l, d), lambda i: (i, 0, 0)),
        out_shape=jax.ShapeDtypeStruct((b, l, d), jnp.float32),
    )(gathered, seg_ids, pos_tab, seg_tab, gamma, beta)


def kernel(input_ids, segment_ids, token_table, position_table, segment_table,
           ln_gamma, ln_beta):
    b, l = input_ids.shape
    d = token_table.shape[1]
    flat_ids = input_ids.reshape(b * l).astype(jnp.int32)
    gathered = _sc_gather(token_table, flat_ids)
    return _tc_epilogue(
        gathered.reshape(b, l, d),
        segment_ids.astype(jnp.float32),
        position_table[:l],
        segment_table,
        ln_gamma.reshape(1, d),
        ln_beta.reshape(1, d),
    )


# speedup vs baseline: 11.8315x; 1.0954x over previous
import functools

import jax
import jax.numpy as jnp
from jax import lax
from jax.experimental import pallas as pl
from jax.experimental.pallas import tpu as pltpu
from jax.experimental.pallas import tpu_sc as plsc

EPS = 1e-5


def _sc_gather(table, idx, start=0, count=None, chunk=256, nbuf=3):
    n = idx.shape[0] if count is None else count
    d = table.shape[1]
    info = plsc.get_sparse_core_info()
    nc, ns = info.num_cores, info.num_subcores
    nw = nc * ns
    per_w = n // nw
    while per_w % chunk or chunk % 8:
        chunk -= 8
    n_chunks = per_w // chunk
    assert per_w % chunk == 0 and n % nw == 0

    mesh = plsc.VectorSubcoreMesh(core_axis_name="c", subcore_axis_name="s")

    @functools.partial(
        pl.kernel,
        mesh=mesh,
        out_type=jax.ShapeDtypeStruct((n, d), jnp.float32),
        scratch_types=[
            pltpu.VMEM((per_w,), jnp.int32),
            pltpu.VMEM((nbuf, chunk, d), jnp.float32),
            pltpu.SemaphoreType.DMA,
            [pltpu.SemaphoreType.DMA] * nbuf,
            [pltpu.SemaphoreType.DMA] * nbuf,
        ],
    )
    def k(table_hbm, idx_hbm, out_hbm, idx_v, rows_v, isem, gsems, wsems):
        wid = lax.axis_index("s") * nc + lax.axis_index("c")
        base = wid * per_w
        pltpu.async_copy(idx_hbm.at[pl.ds(start + base, per_w)], idx_v, isem).wait()

        def g_start(c, b):
            pltpu.async_copy(
                table_hbm.at[idx_v.at[pl.ds(c * chunk, chunk)]],
                rows_v.at[b], gsems[b])

        for b in range(min(nbuf, n_chunks)):
            g_start(b, b)
        for c in range(n_chunks):
            b = c % nbuf
            pltpu.make_async_copy(
                table_hbm.at[idx_v.at[pl.ds(c * chunk, chunk)]],
                rows_v.at[b], gsems[b]).wait()
            w = pltpu.async_copy(
                rows_v.at[b], out_hbm.at[pl.ds(base + c * chunk, chunk)],
                wsems[b])
            if c + nbuf < n_chunks:
                w.wait()
                g_start(c + nbuf, b)
        for c in range(max(0, n_chunks - nbuf), n_chunks):
            b = c % nbuf
            pltpu.make_async_copy(
                rows_v.at[b], out_hbm.at[pl.ds(base + c * chunk, chunk)],
                wsems[b]).wait()

    return k(table, idx)


def _tc_epilogue(gathered, seg_ids, pos_tab, seg_tab, gamma, beta,
                 prev=None, row_off=0, out_rows=None):
    bs, l, d = gathered.shape
    if out_rows is None:
        out_rows = bs
    blk = 16
    grid = (bs // blk,)
    blk_off = row_off // blk

    def body(g_ref, s_ref, p_ref, st_ref, ga_ref, be_ref, o_ref):
        x = g_ref[...]
        segf = s_ref[...]
        st = st_ref[...]
        p0 = p_ref[...] + st[0][None, :]
        sd = st[1] - st[0]
        emb = x + p0[None, :, :] + segf[..., None] * sd[None, None, :]
        s1 = jnp.sum(emb, axis=-1, keepdims=True)
        s2 = jnp.sum(emb * emb, axis=-1, keepdims=True)
        mean = s1 * (1.0 / d)
        var = s2 * (1.0 / d) - mean * mean
        r = lax.rsqrt(var + EPS)
        o_ref[...] = (emb - mean) * r * ga_ref[0][None, None, :] + be_ref[0][None, None, :]

    in_specs = [
        pl.BlockSpec((blk, l, d), lambda i: (i, 0, 0)),
        pl.BlockSpec((blk, l), lambda i: (i + blk_off, 0)),
        pl.BlockSpec((l, d), lambda i: (0, 0)),
        pl.BlockSpec((2, d), lambda i: (0, 0)),
        pl.BlockSpec((1, d), lambda i: (0, 0)),
        pl.BlockSpec((1, d), lambda i: (0, 0)),
    ]
    args = [gathered, seg_ids, pos_tab, seg_tab, gamma, beta]
    kwargs = {}
    if prev is not None:
        def body_p(_, *refs):
            body(*refs)
        fn = body_p
        in_specs = [pl.BlockSpec(memory_space=pl.ANY)] + in_specs
        args = [prev] + args
        kwargs["input_output_aliases"] = {0: 0}
    else:
        fn = body
    return pl.pallas_call(
        fn,
        grid=grid,
        in_specs=in_specs,
        out_specs=pl.BlockSpec((blk, l, d), lambda i: (i + blk_off, 0, 0)),
        out_shape=jax.ShapeDtypeStruct((out_rows, l, d), jnp.float32),
        **kwargs,
    )(*args)


def kernel(input_ids, segment_ids, token_table, position_table, segment_table,
           ln_gamma, ln_beta):
    b, l = input_ids.shape
    d = token_table.shape[1]
    splits = 4
    bs = b // splits
    flat_ids = input_ids.reshape(b * l).astype(jnp.int32)
    segf = segment_ids.astype(jnp.float32)
    pos = position_table[:l]
    gamma = ln_gamma.reshape(1, d)
    beta = ln_beta.reshape(1, d)
    pieces = [
        _sc_gather(token_table, flat_ids, start=i * bs * l, count=bs * l)
        for i in range(splits)
    ]
    out = None
    for i in range(splits):
        out = _tc_epilogue(
            pieces[i].reshape(bs, l, d), segf, pos, segment_table, gamma, beta,
            prev=out, row_off=i * bs, out_rows=b,
        )
    return out
